# baseline (speedup 1.0000x reference)
import jax
import jax.numpy as jnp
from jax.experimental import pallas as pl
from jax.experimental.pallas import tpu as pltpu

KTOP = 16
DEPTH = 4


def _fused_kernel(s_ref, xr_ref, xc_ref, q_ref, vals_ref, idx_ref, x2c_ref):
    xr = xr_ref[0]
    xc = xc_ref[0]
    q = q_ref[0]
    s = s_ref[0]

    @pl.when(pl.program_id(1) == 0)
    def _norms():
        x2c_ref[...] = jnp.sum(xc * xc, axis=1)[None, :]

    dot = jax.lax.dot_general(
        xr, xc, (((1,), (1,)), ((), ())),
        preferred_element_type=jnp.float32,
        precision=jax.lax.Precision.DEFAULT,
    )
    x2r = jnp.sum(xr * xr, axis=1)[:, None]
    x2c = x2c_ref[...]
    d2 = jnp.maximum(x2r + x2c - 2.0 * dot, 0.0)

    score = jnp.log(-jnp.log(q)) - d2 * s

    r, n = score.shape
    lanes = 128
    nchunks = n // lanes
    neg = -jnp.inf

    chunks = [score[:, c * lanes:(c + 1) * lanes] for c in range(nchunks)]
    lane_iota = jax.lax.broadcasted_iota(jnp.int32, (r, lanes), 1).astype(
        jnp.float32
    )
    cand_v = []
    cand_i = []
    for _ in range(DEPTH):
        m = chunks[0]
        for c in range(1, nchunks):
            m = jnp.maximum(m, chunks[c])
        a = jnp.full((r, lanes), float(nchunks), jnp.float32)
        for c in range(nchunks - 1, -1, -1):
            a = jnp.where(chunks[c] == m, float(c), a)
        cand_v.append(m)
        cand_i.append(a * lanes + lane_iota)
        for c in range(nchunks):
            chunks[c] = jnp.where(
                (chunks[c] == m) & (a == c), neg, chunks[c]
            )

    hidden = chunks[0]
    for c in range(1, nchunks):
        hidden = jnp.maximum(hidden, chunks[c])

    h_v, s1v, s2v, s3v = [jnp.transpose(v) for v in cand_v]
    h_i, s1i, s2i, s3i = [jnp.transpose(i) for i in cand_i]
    vals = []
    idxs = []
    for _ in range(KTOP):
        m = jnp.max(h_v, axis=0, keepdims=True)
        gidx = jnp.min(
            jnp.where(h_v == m, h_i, n), axis=0, keepdims=True
        )
        vals.append(m)
        idxs.append(gidx)
        sel = h_i == gidx
        h_v = jnp.where(sel, s1v, h_v)
        h_i = jnp.where(sel, s1i, h_i)
        s1v = jnp.where(sel, s2v, s1v)
        s1i = jnp.where(sel, s2i, s1i)
        s2v = jnp.where(sel, s3v, s2v)
        s2i = jnp.where(sel, s3i, s2i)
        s3v = jnp.where(sel, neg, s3v)
        s3i = jnp.where(sel, n, s3i)
    out_v = jnp.concatenate(vals, axis=0)
    out_i = jnp.concatenate(idxs, axis=0)
    vals_ref[0] = jnp.transpose(out_v)
    idx_ref[0] = jnp.transpose(out_i).astype(jnp.int32)

    v16 = out_v[KTOP - 1][:, None]
    bad = jnp.any((jnp.transpose(h_v) == neg) & (hidden >= v16))

    @pl.when(bad)
    def _fallback():
        iota = jax.lax.broadcasted_iota(jnp.int32, (r, n), 1)
        cur = score
        fvals = []
        fidxs = []
        for _ in range(KTOP):
            fm = jnp.max(cur, axis=1, keepdims=True)
            fi = jnp.min(jnp.where(cur == fm, iota, n), axis=1, keepdims=True)
            fvals.append(fm)
            fidxs.append(fi)
            cur = jnp.where(iota == fi, neg, cur)
        vals_ref[0] = jnp.concatenate(fvals, axis=1)
        idx_ref[0] = jnp.concatenate(fidxs, axis=1)


def _topk(x, s, q, row_block):
    b, n, d = x.shape
    grid = (b, n // row_block)
    vals, idx = pl.pallas_call(
        _fused_kernel,
        grid=grid,
        in_specs=[
            pl.BlockSpec(memory_space=pltpu.SMEM),
            pl.BlockSpec((1, row_block, d), lambda bi, ri: (bi, ri, 0)),
            pl.BlockSpec((1, n, d), lambda bi, ri: (bi, 0, 0)),
            pl.BlockSpec((1, row_block, n), lambda bi, ri: (bi, ri, 0)),
        ],
        out_specs=[
            pl.BlockSpec((1, row_block, KTOP), lambda bi, ri: (bi, ri, 0)),
            pl.BlockSpec((1, row_block, KTOP), lambda bi, ri: (bi, ri, 0)),
        ],
        out_shape=[
            jax.ShapeDtypeStruct((b, n, KTOP), jnp.float32),
            jax.ShapeDtypeStruct((b, n, KTOP), jnp.int32),
        ],
        compiler_params=pltpu.CompilerParams(
            dimension_semantics=("arbitrary", "arbitrary"),
        ),
        scratch_shapes=[pltpu.VMEM((1, n), jnp.float32)],
    )(s, x, x, q)
    return vals, idx


def kernel(x, A, temperature, q):
    b, n, d = x.shape
    s = jnp.exp(jnp.clip(temperature, -5.0, 5.0)).reshape(1)
    logprobs, indices = _topk(x, s, q, 256)

    rows = jnp.broadcast_to(
        jnp.arange(n, dtype=indices.dtype)[None, :, None], (b, n, KTOP)
    )
    edges = jnp.stack((indices.reshape(b, -1), rows.reshape(b, -1)), axis=-2)
    offset = (jnp.arange(b, dtype=indices.dtype) * n)[:, None, None]
    edges_hat = jnp.transpose(edges + offset, (1, 0, 2)).reshape(2, -1)
    return (x, edges_hat, logprobs)


# speedup vs baseline: 1.0331x; 1.0331x over previous
import jax
import jax.numpy as jnp
from jax.experimental import pallas as pl
from jax.experimental.pallas import tpu as pltpu

KTOP = 16
DEPTH = 4


def _fused_kernel(s_ref, xr_ref, xc_ref, q_ref, vals_ref, idx_ref, x2c_ref):
    xr = xr_ref[0]
    xc = xc_ref[0]
    q = q_ref[0]
    s = s_ref[0]

    @pl.when(pl.program_id(1) == 0)
    def _norms():
        x2c_ref[...] = jnp.sum(xc * xc, axis=1)[None, :]

    dot = jax.lax.dot_general(
        xr, xc, (((1,), (1,)), ((), ())),
        preferred_element_type=jnp.float32,
        precision=jax.lax.Precision.DEFAULT,
    )
    x2r = jnp.sum(xr * xr, axis=1)[:, None]
    x2c = x2c_ref[...]
    d2 = jnp.maximum(x2r + x2c - 2.0 * dot, 0.0)

    score = jnp.log(-jnp.log(q)) - d2 * s

    r, n = score.shape
    lanes = 128
    nchunks = n // lanes
    neg = -jnp.inf

    chunks = [score[:, c * lanes:(c + 1) * lanes] for c in range(nchunks)]
    lane_iota = jax.lax.broadcasted_iota(jnp.int32, (r, lanes), 1).astype(
        jnp.float32
    )
    cand_v = []
    cand_i = []
    for _ in range(DEPTH):
        m = chunks[0]
        for c in range(1, nchunks):
            m = jnp.maximum(m, chunks[c])
        a = jnp.full((r, lanes), float(nchunks), jnp.float32)
        for c in range(nchunks - 1, -1, -1):
            a = jnp.where(chunks[c] == m, float(c), a)
        cand_v.append(m)
        cand_i.append(a * lanes + lane_iota)
        for c in range(nchunks):
            chunks[c] = jnp.where(
                (chunks[c] == m) & (a == c), neg, chunks[c]
            )

    hidden = chunks[0]
    for c in range(1, nchunks):
        hidden = jnp.maximum(hidden, chunks[c])

    h_v, s1v, s2v, s3v = [jnp.transpose(v) for v in cand_v]
    h_i, s1i, s2i, s3i = [jnp.transpose(i) for i in cand_i]
    vals = []
    idxs = []
    for _ in range(KTOP):
        m = jnp.max(h_v, axis=0, keepdims=True)
        gidx = jnp.min(
            jnp.where(h_v == m, h_i, n), axis=0, keepdims=True
        )
        vals.append(m)
        idxs.append(gidx)
        sel = h_i == gidx
        h_v = jnp.where(sel, s1v, h_v)
        h_i = jnp.where(sel, s1i, h_i)
        s1v = jnp.where(sel, s2v, s1v)
        s1i = jnp.where(sel, s2i, s1i)
        s2v = jnp.where(sel, s3v, s2v)
        s2i = jnp.where(sel, s3i, s2i)
        s3v = jnp.where(sel, neg, s3v)
        s3i = jnp.where(sel, n, s3i)
    out_v = jnp.concatenate(vals, axis=0)
    out_i = jnp.concatenate(idxs, axis=0)
    vals_ref[0] = jnp.transpose(out_v)
    idx_ref[0] = jnp.transpose(out_i).astype(jnp.int32)

    v16 = out_v[KTOP - 1][:, None]
    bad = jnp.any((jnp.transpose(h_v) == neg) & (hidden >= v16))

    @pl.when(bad)
    def _fallback():
        iota = jax.lax.broadcasted_iota(jnp.int32, (r, n), 1)
        cur = score
        fvals = []
        fidxs = []
        for _ in range(KTOP):
            fm = jnp.max(cur, axis=1, keepdims=True)
            fi = jnp.min(jnp.where(cur == fm, iota, n), axis=1, keepdims=True)
            fvals.append(fm)
            fidxs.append(fi)
            cur = jnp.where(iota == fi, neg, cur)
        vals_ref[0] = jnp.concatenate(fvals, axis=1)
        idx_ref[0] = jnp.concatenate(fidxs, axis=1)


def _topk(x, s, q, row_block):
    b, n, d = x.shape
    grid = (b, n // row_block)
    vals, idx = pl.pallas_call(
        _fused_kernel,
        grid=grid,
        in_specs=[
            pl.BlockSpec(memory_space=pltpu.SMEM),
            pl.BlockSpec((1, row_block, d), lambda bi, ri: (bi, ri, 0)),
            pl.BlockSpec((1, n, d), lambda bi, ri: (bi, 0, 0)),
            pl.BlockSpec((1, row_block, n), lambda bi, ri: (bi, ri, 0)),
        ],
        out_specs=[
            pl.BlockSpec((1, row_block, KTOP), lambda bi, ri: (bi, ri, 0)),
            pl.BlockSpec((1, row_block, KTOP), lambda bi, ri: (bi, ri, 0)),
        ],
        out_shape=[
            jax.ShapeDtypeStruct((b, n, KTOP), jnp.float32),
            jax.ShapeDtypeStruct((b, n, KTOP), jnp.int32),
        ],
        compiler_params=pltpu.CompilerParams(
            dimension_semantics=("arbitrary", "arbitrary"),
        ),
        scratch_shapes=[pltpu.VMEM((1, n), jnp.float32)],
    )(s, x, x, q)
    return vals, idx


def kernel(x, A, temperature, q):
    b, n, d = x.shape
    s = jnp.exp(jnp.clip(temperature, -5.0, 5.0)).reshape(1)
    logprobs, indices = _topk(x, s, q, 512)

    rows = jnp.broadcast_to(
        jnp.arange(n, dtype=indices.dtype)[None, :, None], (b, n, KTOP)
    )
    edges = jnp.stack((indices.reshape(b, -1), rows.reshape(b, -1)), axis=-2)
    offset = (jnp.arange(b, dtype=indices.dtype) * n)[:, None, None]
    edges_hat = jnp.transpose(edges + offset, (1, 0, 2)).reshape(2, -1)
    return (x, edges_hat, logprobs)


# speedup vs baseline: 1.0793x; 1.0448x over previous
import jax
import jax.numpy as jnp
from jax.experimental import pallas as pl
from jax.experimental.pallas import tpu as pltpu

KTOP = 16
DEPTH = 4


def _fused_kernel(s_ref, xr_ref, xc_ref, q_ref, vals_ref, idx_ref):
    xr = xr_ref[0]
    xc = xc_ref[0]
    q = q_ref[0]
    s = s_ref[0]

    dot = jax.lax.dot_general(
        xr, xc, (((1,), (1,)), ((), ())),
        preferred_element_type=jnp.float32,
        precision=jax.lax.Precision.DEFAULT,
    )
    x2r = jnp.sum(xr * xr, axis=1)[:, None]
    x2c = jnp.sum(xc * xc, axis=1)[None, :]
    d2 = jnp.maximum(x2r + x2c - 2.0 * dot, 0.0)

    score = jnp.log(-jnp.log(q)) - d2 * s

    r, n = score.shape
    lanes = 128
    nchunks = n // lanes
    neg = -jnp.inf

    chunks = [score[:, c * lanes:(c + 1) * lanes] for c in range(nchunks)]
    lane_iota = jax.lax.broadcasted_iota(jnp.int32, (r, lanes), 1).astype(
        jnp.float32
    )
    cand_v = []
    cand_i = []
    for _ in range(DEPTH):
        m = chunks[0]
        for c in range(1, nchunks):
            m = jnp.maximum(m, chunks[c])
        a = jnp.full((r, lanes), float(nchunks), jnp.float32)
        for c in range(nchunks - 1, -1, -1):
            a = jnp.where(chunks[c] == m, float(c), a)
        cand_v.append(m)
        cand_i.append(a * lanes + lane_iota)
        for c in range(nchunks):
            chunks[c] = jnp.where(
                (chunks[c] == m) & (a == c), neg, chunks[c]
            )

    hidden = chunks[0]
    for c in range(1, nchunks):
        hidden = jnp.maximum(hidden, chunks[c])

    h_v, s1v, s2v, s3v = [jnp.transpose(v) for v in cand_v]
    h_i, s1i, s2i, s3i = [jnp.transpose(i) for i in cand_i]
    vals = []
    idxs = []
    for _ in range(KTOP):
        m = jnp.max(h_v, axis=0, keepdims=True)
        gidx = jnp.min(
            jnp.where(h_v == m, h_i, n), axis=0, keepdims=True
        )
        vals.append(m)
        idxs.append(gidx)
        sel = h_i == gidx
        h_v = jnp.where(sel, s1v, h_v)
        h_i = jnp.where(sel, s1i, h_i)
        s1v = jnp.where(sel, s2v, s1v)
        s1i = jnp.where(sel, s2i, s1i)
        s2v = jnp.where(sel, s3v, s2v)
        s2i = jnp.where(sel, s3i, s2i)
        s3v = jnp.where(sel, neg, s3v)
        s3i = jnp.where(sel, n, s3i)
    out_v = jnp.concatenate(vals, axis=0)
    out_i = jnp.concatenate(idxs, axis=0)
    vals_ref[0] = jnp.transpose(out_v)
    idx_ref[0] = jnp.transpose(out_i).astype(jnp.int32)

    v16 = out_v[KTOP - 1][:, None]
    bad = jnp.any((jnp.transpose(h_v) == neg) & (hidden >= v16))

    @pl.when(bad)
    def _fallback():
        iota = jax.lax.broadcasted_iota(jnp.int32, (r, n), 1)
        cur = score
        fvals = []
        fidxs = []
        for _ in range(KTOP):
            fm = jnp.max(cur, axis=1, keepdims=True)
            fi = jnp.min(jnp.where(cur == fm, iota, n), axis=1, keepdims=True)
            fvals.append(fm)
            fidxs.append(fi)
            cur = jnp.where(iota == fi, neg, cur)
        vals_ref[0] = jnp.concatenate(fvals, axis=1)
        idx_ref[0] = jnp.concatenate(fidxs, axis=1)


def _topk(x, s, q, row_block):
    b, n, d = x.shape
    grid = (b, n // row_block)
    vals, idx = pl.pallas_call(
        _fused_kernel,
        grid=grid,
        in_specs=[
            pl.BlockSpec(memory_space=pltpu.SMEM),
            pl.BlockSpec((1, row_block, d), lambda bi, ri: (bi, ri, 0)),
            pl.BlockSpec((1, n, d), lambda bi, ri: (bi, 0, 0)),
            pl.BlockSpec((1, row_block, n), lambda bi, ri: (bi, ri, 0)),
        ],
        out_specs=[
            pl.BlockSpec((1, row_block, KTOP), lambda bi, ri: (bi, ri, 0)),
            pl.BlockSpec((1, row_block, KTOP), lambda bi, ri: (bi, ri, 0)),
        ],
        out_shape=[
            jax.ShapeDtypeStruct((b, n, KTOP), jnp.float32),
            jax.ShapeDtypeStruct((b, n, KTOP), jnp.int32),
        ],
        compiler_params=pltpu.CompilerParams(
            dimension_semantics=("arbitrary", "arbitrary"),
        ),
    )(s, x, x, q)
    return vals, idx


def kernel(x, A, temperature, q):
    b, n, d = x.shape
    s = jnp.exp(jnp.clip(temperature, -5.0, 5.0)).reshape(1)
    logprobs, indices = _topk(x, s, q, 512)

    rows = jnp.broadcast_to(
        jnp.arange(n, dtype=indices.dtype)[None, :, None], (b, n, KTOP)
    )
    edges = jnp.stack((indices.reshape(b, -1), rows.reshape(b, -1)), axis=-2)
    offset = (jnp.arange(b, dtype=indices.dtype) * n)[:, None, None]
    edges_hat = jnp.transpose(edges + offset, (1, 0, 2)).reshape(2, -1)
    return (x, edges_hat, logprobs)


# speedup vs baseline: 1.1399x; 1.0561x over previous
import jax
import jax.numpy as jnp
from jax.experimental import pallas as pl
from jax.experimental.pallas import tpu as pltpu

KTOP = 16
DEPTH = 4


def _fused_kernel(s_ref, xr_ref, xc_ref, q_ref, vals_ref, idx_ref):
    xr = xr_ref[0]
    xc = xc_ref[0]
    q = q_ref[0]
    s = s_ref[0]

    dot = jax.lax.dot_general(
        xr, xc, (((1,), (1,)), ((), ())),
        preferred_element_type=jnp.float32,
        precision=jax.lax.Precision.DEFAULT,
    )
    x2r = jnp.sum(xr * xr, axis=1)[:, None]
    x2c = jnp.sum(xc * xc, axis=1)[None, :]
    d2 = jnp.maximum(x2r + x2c - 2.0 * dot, 0.0)

    score = jnp.log(-jnp.log(q)) - d2 * s

    r, n = score.shape
    lanes = 128
    nchunks = n // lanes
    neg = -jnp.inf

    chunks = [score[:, c * lanes:(c + 1) * lanes] for c in range(nchunks)]
    lane_iota = jax.lax.broadcasted_iota(jnp.int32, (r, lanes), 1).astype(
        jnp.float32
    )
    cand_v = []
    cand_i = []
    for _ in range(DEPTH):
        m = chunks[0]
        for c in range(1, nchunks):
            m = jnp.maximum(m, chunks[c])
        a = jnp.full((r, lanes), float(nchunks), jnp.float32)
        for c in range(nchunks - 1, -1, -1):
            a = jnp.where(chunks[c] == m, float(c), a)
        cand_v.append(m)
        cand_i.append(a * lanes + lane_iota)
        for c in range(nchunks):
            chunks[c] = jnp.where(a == c, neg, chunks[c])

    hidden = chunks[0]
    for c in range(1, nchunks):
        hidden = jnp.maximum(hidden, chunks[c])

    h_v, s1v, s2v, s3v = [jnp.transpose(v) for v in cand_v]
    h_i, s1i, s2i, s3i = [jnp.transpose(i) for i in cand_i]
    vals = []
    idxs = []
    for _ in range(KTOP):
        m = jnp.max(h_v, axis=0, keepdims=True)
        gidx = jnp.min(
            jnp.where(h_v == m, h_i, n), axis=0, keepdims=True
        )
        vals.append(m)
        idxs.append(gidx)
        sel = h_i == gidx
        h_v = jnp.where(sel, s1v, h_v)
        h_i = jnp.where(sel, s1i, h_i)
        s1v = jnp.where(sel, s2v, s1v)
        s1i = jnp.where(sel, s2i, s1i)
        s2v = jnp.where(sel, s3v, s2v)
        s2i = jnp.where(sel, s3i, s2i)
        s3v = jnp.where(sel, neg, s3v)
        s3i = jnp.where(sel, n, s3i)
    out_v = jnp.concatenate(vals, axis=0)
    out_i = jnp.concatenate(idxs, axis=0)
    vals_ref[0] = jnp.transpose(out_v)
    idx_ref[0] = jnp.transpose(out_i).astype(jnp.int32)

    v16 = out_v[KTOP - 1][:, None]
    bad = jnp.any((jnp.transpose(h_v) == neg) & (hidden >= v16))

    @pl.when(bad)
    def _fallback():
        iota = jax.lax.broadcasted_iota(jnp.int32, (r, n), 1)
        cur = score
        fvals = []
        fidxs = []
        for _ in range(KTOP):
            fm = jnp.max(cur, axis=1, keepdims=True)
            fi = jnp.min(jnp.where(cur == fm, iota, n), axis=1, keepdims=True)
            fvals.append(fm)
            fidxs.append(fi)
            cur = jnp.where(iota == fi, neg, cur)
        vals_ref[0] = jnp.concatenate(fvals, axis=1)
        idx_ref[0] = jnp.concatenate(fidxs, axis=1)


def _topk(x, s, q, row_block):
    b, n, d = x.shape
    grid = (b, n // row_block)
    vals, idx = pl.pallas_call(
        _fused_kernel,
        grid=grid,
        in_specs=[
            pl.BlockSpec(memory_space=pltpu.SMEM),
            pl.BlockSpec((1, row_block, d), lambda bi, ri: (bi, ri, 0)),
            pl.BlockSpec((1, n, d), lambda bi, ri: (bi, 0, 0)),
            pl.BlockSpec((1, row_block, n), lambda bi, ri: (bi, ri, 0)),
        ],
        out_specs=[
            pl.BlockSpec((1, row_block, KTOP), lambda bi, ri: (bi, ri, 0)),
            pl.BlockSpec((1, row_block, KTOP), lambda bi, ri: (bi, ri, 0)),
        ],
        out_shape=[
            jax.ShapeDtypeStruct((b, n, KTOP), jnp.float32),
            jax.ShapeDtypeStruct((b, n, KTOP), jnp.int32),
        ],
        compiler_params=pltpu.CompilerParams(
            dimension_semantics=("arbitrary", "arbitrary"),
        ),
    )(s, x, x, q)
    return vals, idx


def kernel(x, A, temperature, q):
    b, n, d = x.shape
    s = jnp.exp(jnp.clip(temperature, -5.0, 5.0)).reshape(1)
    logprobs, indices = _topk(x, s, q, 512)

    rows = jnp.broadcast_to(
        jnp.arange(n, dtype=indices.dtype)[None, :, None], (b, n, KTOP)
    )
    edges = jnp.stack((indices.reshape(b, -1), rows.reshape(b, -1)), axis=-2)
    offset = (jnp.arange(b, dtype=indices.dtype) * n)[:, None, None]
    edges_hat = jnp.transpose(edges + offset, (1, 0, 2)).reshape(2, -1)
    return (x, edges_hat, logprobs)


# speedup vs baseline: 1.1617x; 1.0191x over previous
import jax
import jax.numpy as jnp
from jax.experimental import pallas as pl
from jax.experimental.pallas import tpu as pltpu

KTOP = 16
DEPTH = 4


def _fused_kernel(s_ref, xr_ref, xc_ref, q_ref, vals_ref, idx_ref):
    xr = xr_ref[0]
    xc = xc_ref[0]
    q = q_ref[0]
    s = s_ref[0]

    dot = jax.lax.dot_general(
        xr, xc, (((1,), (1,)), ((), ())),
        preferred_element_type=jnp.float32,
        precision=jax.lax.Precision.DEFAULT,
    )
    x2r = jnp.sum(xr * xr, axis=1)[:, None]
    x2c = jnp.sum(xc * xc, axis=1)[None, :]
    d2 = jnp.maximum(x2r + x2c - 2.0 * dot, 0.0)

    score = jnp.log(-jnp.log(q)) - d2 * s

    r, n = score.shape
    lanes = 128
    nchunks = n // lanes
    neg = -jnp.inf

    chunks = [score[:, c * lanes:(c + 1) * lanes] for c in range(nchunks)]
    lane_iota = jax.lax.broadcasted_iota(jnp.int32, (r, lanes), 1).astype(
        jnp.float32
    )
    cand_v = []
    cand_i = []
    for _ in range(DEPTH):
        m = chunks[0]
        for c in range(1, nchunks):
            m = jnp.maximum(m, chunks[c])
        a = jnp.full((r, lanes), float(nchunks), jnp.float32)
        for c in range(nchunks - 1, -1, -1):
            a = jnp.where(chunks[c] == m, float(c), a)
        cand_v.append(m)
        cand_i.append(a * lanes + lane_iota)
        for c in range(nchunks):
            chunks[c] = jnp.where(a == c, neg, chunks[c])

    hidden = chunks[0]
    for c in range(1, nchunks):
        hidden = jnp.maximum(hidden, chunks[c])

    h_v, s1v, s2v, s3v = [jnp.transpose(v) for v in cand_v]
    h_i, s1i, s2i, s3i = [jnp.transpose(i) for i in cand_i]
    vals = []
    idxs = []
    for _ in range(KTOP):
        m = jnp.max(h_v, axis=0, keepdims=True)
        gidx = jnp.min(
            jnp.where(h_v == m, h_i, n), axis=0, keepdims=True
        )
        vals.append(m)
        idxs.append(gidx)
        sel = h_i == gidx
        h_v = jnp.where(sel, s1v, h_v)
        h_i = jnp.where(sel, s1i, h_i)
        s1v = jnp.where(sel, s2v, s1v)
        s1i = jnp.where(sel, s2i, s1i)
        s2v = jnp.where(sel, s3v, s2v)
        s2i = jnp.where(sel, s3i, s2i)
        s3v = jnp.where(sel, neg, s3v)
    out_v = jnp.concatenate(vals, axis=0)
    out_i = jnp.concatenate(idxs, axis=0)
    vals_ref[0] = jnp.transpose(out_v)
    idx_ref[0] = jnp.transpose(out_i).astype(jnp.int32)

    v16 = out_v[KTOP - 1][:, None]
    bad = jnp.any((jnp.transpose(h_v) == neg) & (hidden >= v16))

    @pl.when(bad)
    def _fallback():
        iota = jax.lax.broadcasted_iota(jnp.int32, (r, n), 1)
        cur = score
        fvals = []
        fidxs = []
        for _ in range(KTOP):
            fm = jnp.max(cur, axis=1, keepdims=True)
            fi = jnp.min(jnp.where(cur == fm, iota, n), axis=1, keepdims=True)
            fvals.append(fm)
            fidxs.append(fi)
            cur = jnp.where(iota == fi, neg, cur)
        vals_ref[0] = jnp.concatenate(fvals, axis=1)
        idx_ref[0] = jnp.concatenate(fidxs, axis=1)


def _topk(x, s, q, row_block):
    b, n, d = x.shape
    grid = (b, n // row_block)
    vals, idx = pl.pallas_call(
        _fused_kernel,
        grid=grid,
        in_specs=[
            pl.BlockSpec(memory_space=pltpu.SMEM),
            pl.BlockSpec((1, row_block, d), lambda bi, ri: (bi, ri, 0)),
            pl.BlockSpec((1, n, d), lambda bi, ri: (bi, 0, 0)),
            pl.BlockSpec((1, row_block, n), lambda bi, ri: (bi, ri, 0)),
        ],
        out_specs=[
            pl.BlockSpec((1, row_block, KTOP), lambda bi, ri: (bi, ri, 0)),
            pl.BlockSpec((1, row_block, KTOP), lambda bi, ri: (bi, ri, 0)),
        ],
        out_shape=[
            jax.ShapeDtypeStruct((b, n, KTOP), jnp.float32),
            jax.ShapeDtypeStruct((b, n, KTOP), jnp.int32),
        ],
        compiler_params=pltpu.CompilerParams(
            dimension_semantics=("arbitrary", "arbitrary"),
        ),
    )(s, x, x, q)
    return vals, idx


def kernel(x, A, temperature, q):
    b, n, d = x.shape
    s = jnp.exp(jnp.clip(temperature, -5.0, 5.0)).reshape(1)
    logprobs, indices = _topk(x, s, q, 512)

    rows = jnp.broadcast_to(
        jnp.arange(n, dtype=indices.dtype)[None, :, None], (b, n, KTOP)
    )
    edges = jnp.stack((indices.reshape(b, -1), rows.reshape(b, -1)), axis=-2)
    offset = (jnp.arange(b, dtype=indices.dtype) * n)[:, None, None]
    edges_hat = jnp.transpose(edges + offset, (1, 0, 2)).reshape(2, -1)
    return (x, edges_hat, logprobs)
